# baseline (speedup 1.0000x reference)
import functools

import numpy as np
import jax
import jax.numpy as jnp
from jax import lax
from jax.experimental import pallas as pl
from jax.experimental.pallas import tpu as pltpu
from jax.experimental.pallas import tpu_sc as plsc

_VAR = np.array([(0, 1)] * 10 + [(0, 2)] * 15, dtype=np.int32)
_MEM = np.array(
    [(1, 0), (1, 1), (1, 2), (1, 3), (1, 4), (3, 4), (3, 3), (3, 2), (3, 1),
     (3, 0), (1, 0), (1, 1), (1, 2), (1, 3), (1, 4), (2, 0), (2, 1), (2, 2),
     (2, 3), (2, 4), (3, 0), (3, 1), (3, 2), (3, 3), (3, 4)], dtype=np.int32)
_FLAT = _VAR * 5 + _MEM
_FA = tuple(int(v) for v in _FLAT[:, 0])
_FB = tuple(int(v) for v in _FLAT[:, 1])
_USED = tuple(sorted(set(_FA) | set(_FB)))

_N = 1_000_000
_NF = 15
_NR = 25
_CH = 800
_NCHUNK = _N // _CH
_NW = 32
_GROUPS = _CH // 16
_KMAX = -(-_NCHUNK // _NW)

def _sc_body(x_hbm, o_hbm, xbuf, obuf):
    wid = lax.axis_index("s") * 2 + lax.axis_index("c")
    lanes = lax.iota(jnp.int32, 16)
    la15 = lanes * _NF
    la25 = lanes * _NR

    def chunk_step(k, carry):
        chunk = k * _NW + wid

        @pl.when(chunk < _NCHUNK)
        def _():
            xoff = pl.multiple_of(chunk * (_CH * _NF), 8)
            ooff = pl.multiple_of(chunk * (_CH * _NR), 8)
            pltpu.sync_copy(x_hbm.at[pl.ds(xoff, _CH * _NF)], xbuf)

            def group_step(g, gcarry):
                ib = g * (16 * _NF)
                ob = g * (16 * _NR)
                feats = {}
                for f in _USED:
                    feats[f] = plsc.load_gather(xbuf, [la15 + (ib + f)])
                for r in range(_NR):
                    v = jnp.minimum(feats[_FA[r]], feats[_FB[r]])
                    plsc.store_scatter(obuf, [la25 + (ob + r)], v)
                return gcarry

            lax.fori_loop(0, _GROUPS, group_step, 0)
            pltpu.sync_copy(obuf, o_hbm.at[pl.ds(ooff, _CH * _NR)])

        return carry

    lax.fori_loop(0, _KMAX, chunk_step, 0)


@functools.cache
def _sc_run():
    mesh = plsc.VectorSubcoreMesh(
        core_axis_name="c", subcore_axis_name="s",
        num_cores=2, num_subcores=16)
    return pl.kernel(
        _sc_body,
        out_type=jax.ShapeDtypeStruct((_N * _NR,), jnp.float32),
        mesh=mesh,
        compiler_params=pltpu.CompilerParams(needs_layout_passes=False),
        scratch_types=[
            pltpu.VMEM((_CH * _NF,), jnp.float32),
            pltpu.VMEM((_CH * _NR,), jnp.float32),
        ],
    )


@jax.jit
def kernel(x):
    out = _sc_run()(x.reshape(_N * _NF))
    return out.reshape(_N, _NR)
